# baseline (speedup 1.0000x reference)
import functools

import jax
import jax.numpy as jnp
from jax import lax
from jax.experimental import pallas as pl
from jax.experimental.pallas import tpu as pltpu
from jax.experimental.pallas import tpu_sc as plsc

N = 10000
E = 320000
D = 128
KW = 4

NC = 2
NS = 16
NW = NC * NS
L = 16

CH = 128
EPB = 9984
XTRA = 4
NCHB = EPB // CH
EPMAX = EPB + CH
NBUF = 2
RA = 624
TAIL = N - NS * RA

_mesh = plsc.VectorSubcoreMesh(core_axis_name="c", subcore_axis_name="s")
_sc_params = pltpu.CompilerParams(needs_layout_passes=False)


@functools.partial(
    pl.kernel,
    mesh=_mesh,
    out_type=jax.ShapeDtypeStruct((NW * N,), jnp.float32),
    scratch_types=[
        pltpu.VMEM((2, EPMAX), jnp.int32),
        pltpu.VMEM((N,), jnp.float32),
        pltpu.SemaphoreType.DMA,
        pltpu.SemaphoreType.DMA,
    ],
    compiler_params=_sc_params,
)
def _deg_kernel(ei_hbm, out_hbm, edges_v, hist_v, sem, semx):
    c = lax.axis_index("c")
    s = lax.axis_index("s")
    wid = s * NC + c
    eoff = wid * EPB + jnp.minimum(wid, XTRA) * CH
    nch = NCHB + jnp.where(wid < XTRA, 1, 0)

    cp = pltpu.async_copy(ei_hbm.at[:, pl.ds(eoff, EPB)],
                          edges_v.at[:, pl.ds(0, EPB)], sem)

    @pl.when(wid < XTRA)
    def _():
        pltpu.async_copy(ei_hbm.at[:, pl.ds(eoff + EPB, CH)],
                         edges_v.at[:, pl.ds(EPB, CH)], semx).wait()

    zeros16 = jnp.zeros((L,), jnp.float32)
    ones16 = jnp.ones((L,), jnp.float32)

    def zero_body(i, _):
        hist_v[pl.ds(i * L, L)] = zeros16
        return 0

    lax.fori_loop(0, N // L, zero_body, 0)
    cp.wait()

    def chunk_body(i, _):
        for j in range(CH // L):
            idx = edges_v[1, pl.ds(i * CH + j * L, L)]
            plsc.addupdate_scatter(hist_v, [idx], ones16)
        return 0

    lax.fori_loop(0, nch, chunk_body, 0)
    pltpu.sync_copy(hist_v, out_hbm.at[pl.ds(wid * N, N)])


def _prep_body(hists_ref, x_ref, y_ref, dinv_ref):
    deg = jnp.sum(hists_ref[...], axis=0) + 1.0
    dinv = lax.rsqrt(deg)
    y_ref[...] = x_ref[...] * dinv[:, None]
    dinv_ref[...] = dinv[:, None]


_prep = pl.pallas_call(
    _prep_body,
    out_shape=[
        jax.ShapeDtypeStruct((N, D), jnp.float32),
        jax.ShapeDtypeStruct((N, 1), jnp.float32),
    ],
)


@functools.partial(
    pl.kernel,
    mesh=_mesh,
    out_type=jax.ShapeDtypeStruct((NC, N, D), jnp.float32),
    scratch_types=[
        pltpu.VMEM((EPMAX,), jnp.int32),
        [pltpu.VMEM((2, CH), jnp.int32) for _ in range(NBUF)],
        [pltpu.VMEM((CH, D), jnp.float32) for _ in range(NBUF)],
        pltpu.VMEM_SHARED((N, D), jnp.float32),
        pltpu.SemaphoreType.DMA,
        pltpu.SemaphoreType.DMA,
        [pltpu.SemaphoreType.DMA for _ in range(NBUF)],
        [pltpu.SemaphoreType.DMA for _ in range(NBUF)],
        [pltpu.SemaphoreType.DMA for _ in range(NBUF)],
    ],
    compiler_params=_sc_params,
)
def _agg_kernel(y_hbm, ei_hbm, out_hbm, srcs_v, dbufs, rows,
                accum, si0, si1, sds, sgs, sws):
    c = lax.axis_index("c")
    s = lax.axis_index("s")
    wid = s * NC + c
    base = s * RA
    eoff = wid * EPB + jnp.minimum(wid, XTRA) * CH
    nch = NCHB + jnp.where(wid < XTRA, 1, 0)

    cp_src = pltpu.async_copy(ei_hbm.at[0, pl.ds(eoff, EPB)],
                              srcs_v.at[pl.ds(0, EPB)], si0)

    @pl.when(wid < XTRA)
    def _():
        pltpu.async_copy(ei_hbm.at[0, pl.ds(eoff + EPB, CH)],
                         srcs_v.at[pl.ds(EPB, CH)], si1).wait()

    @pl.when(c == 0)
    def _():
        pltpu.sync_copy(y_hbm.at[pl.ds(base, RA)], accum.at[pl.ds(base, RA)])

        @pl.when(s == 0)
        def _():
            pltpu.sync_copy(y_hbm.at[pl.ds(NS * RA, TAIL)],
                            accum.at[pl.ds(NS * RA, TAIL)])

    @pl.when(c != 0)
    def _():
        zeros16 = jnp.zeros((L,), jnp.float32)

        def zfill(t, _):
            rows[0][t // (D // L), pl.ds((t % (D // L)) * L, L)] = zeros16
            return 0

        lax.fori_loop(0, CH * (D // L), zfill, 0)

        def zcopy(r, _):
            pltpu.sync_copy(rows[0],
                            accum.at[pl.ds(base + r * CH, CH)])
            return 0

        lax.fori_loop(0, RA // CH, zcopy, 0)
        pltpu.sync_copy(rows[0].at[pl.ds(0, RA - (RA // CH) * CH)],
                        accum.at[pl.ds(base + (RA // CH) * CH,
                                       RA - (RA // CH) * CH)])

        @pl.when(s == 0)
        def _():
            pltpu.sync_copy(rows[0].at[pl.ds(0, TAIL)],
                            accum.at[pl.ds(NS * RA, TAIL)])

    cp_src.wait()

    def dst_dma(i, j):
        pltpu.async_copy(ei_hbm.at[:, pl.ds(eoff + i * CH, CH)], dbufs[j],
                         sds[j])

    def dst_wait(j):
        pltpu.make_async_copy(ei_hbm.at[:, pl.ds(0, CH)], dbufs[j],
                              sds[j]).wait()

    def gather(i, j):
        pltpu.async_copy(y_hbm.at[srcs_v.at[pl.ds(i * CH, CH)]], rows[j],
                         sgs[j])

    def gather_wait(j):
        pltpu.make_async_copy(y_hbm.at[pl.ds(0, CH)], rows[j], sgs[j]).wait()

    def scatter_start(j):
        pltpu.async_copy(rows[j], accum.at[dbufs[j].at[1]], sws[j], add=True)

    def scatter_wait(j):
        pltpu.make_async_copy(y_hbm.at[pl.ds(0, CH)], rows[j], sws[j]).wait()

    for j in range(NBUF):
        dst_dma(j, j)
        gather(j, j)

    plsc.subcore_barrier()

    def round_body(p, _):
        for j in range(NBUF):
            i = p * NBUF + j
            jp = (j - 1) % NBUF

            gather_wait(j)
            dst_wait(j)
            scatter_start(j)

            @pl.when(jnp.logical_and(i > 0, i + 1 < nch))
            def _():
                scatter_wait(jp)
                dst_dma(i + 1, jp)
                gather(i + 1, jp)

        return 0

    lax.fori_loop(0, NCHB // NBUF, round_body, 0)

    @pl.when(wid < XTRA)
    def _():
        gather_wait(0)
        dst_wait(0)
        scatter_start(0)

    for j in range(NBUF):
        scatter_wait(j)

    plsc.subcore_barrier()

    pltpu.sync_copy(accum.at[pl.ds(base, RA)],
                    out_hbm.at[c, pl.ds(base, RA)])

    @pl.when(s == 0)
    def _():
        pltpu.sync_copy(accum.at[pl.ds(NS * RA, TAIL)],
                        out_hbm.at[c, pl.ds(NS * RA, TAIL)])


def _out_body(a_ref, dinv_ref, w_ref, b_ref, o_ref):
    t = (a_ref[0] + a_ref[1]) * dinv_ref[...]
    acc = None
    for k in range(KW):
        f = jnp.dot(t, w_ref[k], preferred_element_type=jnp.float32)
        f = jnp.maximum(f + b_ref[k][None, :], 0.0)
        acc = f if acc is None else acc + f
    o_ref[...] = acc


_BN = 400

_out = pl.pallas_call(
    _out_body,
    grid=(N // _BN,),
    in_specs=[
        pl.BlockSpec((NC, _BN, D), lambda i: (0, i, 0)),
        pl.BlockSpec((_BN, 1), lambda i: (i, 0)),
        pl.BlockSpec((KW, D, D), lambda i: (0, 0, 0)),
        pl.BlockSpec((KW, D), lambda i: (0, 0)),
    ],
    out_specs=pl.BlockSpec((_BN, D), lambda i: (i, 0)),
    out_shape=jax.ShapeDtypeStruct((N, D), jnp.float32),
)


@jax.jit
def kernel(x, edge_index, W, b):
    hists = _deg_kernel(edge_index).reshape(NW, N)
    y, dinv = _prep(hists, x)
    partials = _agg_kernel(y, edge_index)
    return _out(partials, dinv, W, b)


# speedup vs baseline: 1.3307x; 1.3307x over previous
import functools

import jax
import jax.numpy as jnp
from jax import lax
from jax.experimental import pallas as pl
from jax.experimental.pallas import tpu as pltpu
from jax.experimental.pallas import tpu_sc as plsc

N = 10000
E = 320000
D = 128
KW = 4

NC = 2
NS = 16
NW = NC * NS
L = 16

CH = 128
GC = 64
EPB = 9984
XTRA = 4
NCHB = EPB // CH
NTOTB = 2 * NCHB
EPMAX = EPB + CH
NGB = 4
RA = 624
TAIL = N - NS * RA

_mesh = plsc.VectorSubcoreMesh(core_axis_name="c", subcore_axis_name="s")
_sc_params = pltpu.CompilerParams(needs_layout_passes=False)


@functools.partial(
    pl.kernel,
    mesh=_mesh,
    out_type=jax.ShapeDtypeStruct((NW * N,), jnp.float32),
    scratch_types=[
        pltpu.VMEM((2, EPMAX), jnp.int32),
        pltpu.VMEM((N,), jnp.float32),
        pltpu.SemaphoreType.DMA,
        pltpu.SemaphoreType.DMA,
    ],
    compiler_params=_sc_params,
)
def _deg_kernel(ei_hbm, out_hbm, edges_v, hist_v, sem, semx):
    c = lax.axis_index("c")
    s = lax.axis_index("s")
    wid = s * NC + c
    eoff = wid * EPB + jnp.minimum(wid, XTRA) * CH
    nch = NCHB + jnp.where(wid < XTRA, 1, 0)

    cp = pltpu.async_copy(ei_hbm.at[:, pl.ds(eoff, EPB)],
                          edges_v.at[:, pl.ds(0, EPB)], sem)

    @pl.when(wid < XTRA)
    def _():
        pltpu.async_copy(ei_hbm.at[:, pl.ds(eoff + EPB, CH)],
                         edges_v.at[:, pl.ds(EPB, CH)], semx).wait()

    zeros16 = jnp.zeros((L,), jnp.float32)
    ones16 = jnp.ones((L,), jnp.float32)

    def zero_body(i, _):
        hist_v[pl.ds(i * L, L)] = zeros16
        return 0

    lax.fori_loop(0, N // L, zero_body, 0)
    cp.wait()

    def chunk_body(i, _):
        for j in range(CH // L):
            idx = edges_v[1, pl.ds(i * CH + j * L, L)]
            plsc.addupdate_scatter(hist_v, [idx], ones16)
        return 0

    lax.fori_loop(0, nch, chunk_body, 0)
    pltpu.sync_copy(hist_v, out_hbm.at[pl.ds(wid * N, N)])


def _prep_body(hists_ref, x_ref, y_ref, dinv_ref):
    deg = jnp.sum(hists_ref[...], axis=0) + 1.0
    dinv = lax.rsqrt(deg)
    y_ref[...] = x_ref[...] * dinv[:, None]
    dinv_ref[...] = dinv[:, None]


_prep = pl.pallas_call(
    _prep_body,
    out_shape=[
        jax.ShapeDtypeStruct((N, D), jnp.float32),
        jax.ShapeDtypeStruct((N, 1), jnp.float32),
    ],
)


@functools.partial(
    pl.kernel,
    mesh=_mesh,
    out_type=jax.ShapeDtypeStruct((NC, N, D), jnp.float32),
    scratch_types=[
        pltpu.VMEM((EPMAX,), jnp.int32),
        [pltpu.VMEM((2, CH), jnp.int32) for _ in range(2)],
        [pltpu.VMEM((GC,), jnp.int32) for _ in range(NGB)],
        [pltpu.VMEM((GC, D), jnp.float32) for _ in range(NGB)],
        pltpu.VMEM_SHARED((N, D), jnp.float32),
        pltpu.SemaphoreType.DMA,
        pltpu.SemaphoreType.DMA,
        [pltpu.SemaphoreType.DMA for _ in range(2)],
        [pltpu.SemaphoreType.DMA for _ in range(NGB)],
        [pltpu.SemaphoreType.DMA for _ in range(NGB)],
    ],
    compiler_params=_sc_params,
)
def _agg_kernel(y_hbm, ei_hbm, out_hbm, srcs_v, dbufs, didx, rows,
                accum, si0, si1, sds, sgs, sws):
    c = lax.axis_index("c")
    s = lax.axis_index("s")
    wid = s * NC + c
    base = s * RA
    eoff = wid * EPB + jnp.minimum(wid, XTRA) * CH
    ntot = NTOTB + jnp.where(wid < XTRA, 2, 0)

    cp_src = pltpu.async_copy(ei_hbm.at[0, pl.ds(eoff, EPB)],
                              srcs_v.at[pl.ds(0, EPB)], si0)

    @pl.when(wid < XTRA)
    def _():
        pltpu.async_copy(ei_hbm.at[0, pl.ds(eoff + EPB, CH)],
                         srcs_v.at[pl.ds(EPB, CH)], si1).wait()

    @pl.when(c == 0)
    def _():
        pltpu.sync_copy(y_hbm.at[pl.ds(base, RA)], accum.at[pl.ds(base, RA)])

        @pl.when(s == 0)
        def _():
            pltpu.sync_copy(y_hbm.at[pl.ds(NS * RA, TAIL)],
                            accum.at[pl.ds(NS * RA, TAIL)])

    @pl.when(c != 0)
    def _():
        zeros16 = jnp.zeros((L,), jnp.float32)

        def zfill(t, _):
            rows[0][t // (D // L), pl.ds((t % (D // L)) * L, L)] = zeros16
            return 0

        lax.fori_loop(0, GC * (D // L), zfill, 0)

        def zcopy(r, _):
            pltpu.sync_copy(rows[0],
                            accum.at[pl.ds(base + r * GC, GC)])
            return 0

        lax.fori_loop(0, RA // GC, zcopy, 0)
        pltpu.sync_copy(rows[0].at[pl.ds(0, RA - (RA // GC) * GC)],
                        accum.at[pl.ds(base + (RA // GC) * GC,
                                       RA - (RA // GC) * GC)])

        @pl.when(s == 0)
        def _():
            pltpu.sync_copy(rows[0].at[pl.ds(0, TAIL)],
                            accum.at[pl.ds(NS * RA, TAIL)])

    cp_src.wait()

    def pair_dma(q, b):
        pltpu.async_copy(ei_hbm.at[:, pl.ds(eoff + q * CH, CH)], dbufs[b],
                         sds[b])

    def pair_wait(b):
        pltpu.make_async_copy(ei_hbm.at[:, pl.ds(0, CH)], dbufs[b],
                              sds[b]).wait()

    def didx_copy(b, h, m):
        for v in range(GC // L):
            didx[m][pl.ds(v * L, L)] = dbufs[b][1, pl.ds(h * GC + v * L, L)]

    def gather(i, m):
        pltpu.async_copy(y_hbm.at[srcs_v.at[pl.ds(i * GC, GC)]], rows[m],
                         sgs[m])

    def gather_wait(m):
        pltpu.make_async_copy(y_hbm.at[pl.ds(0, GC)], rows[m], sgs[m]).wait()

    def scatter_start(m):
        pltpu.async_copy(rows[m], accum.at[didx[m]], sws[m], add=True)

    def scatter_wait(m):
        pltpu.make_async_copy(y_hbm.at[pl.ds(0, GC)], rows[m], sws[m]).wait()

    pair_dma(0, 0)
    pair_dma(1, 1)
    pair_wait(0)
    didx_copy(0, 0, 0)
    didx_copy(0, 1, 1)
    pair_wait(1)
    didx_copy(1, 0, 2)
    didx_copy(1, 1, 3)
    pair_dma(2, 0)
    pair_dma(3, 1)
    for m in range(NGB):
        gather(m, m)

    plsc.subcore_barrier()

    def round_body(p, _):
        for j in range(NGB):
            k = p * NGB + j
            jp = (j - 1) % NGB

            @pl.when(jnp.logical_and(k > 0, k + 3 < ntot))
            def _():
                c = k + 3
                scatter_wait(jp)

                if j % 2 == 1:
                    b = ((j + 3) // 2) % 2
                    pair_wait(b)
                    didx_copy(b, 0, jp)
                else:
                    b = ((j + 2) // 2) % 2
                    didx_copy(b, 1, jp)

                    @pl.when((c + 3) // 2 < ntot // 2)
                    def _():
                        pair_dma((c + 3) // 2, b)

                gather(c, jp)

            gather_wait(j)
            scatter_start(j)

        return 0

    lax.fori_loop(0, NTOTB // NGB, round_body, 0)

    @pl.when(wid < XTRA)
    def _():
        for t in range(2):
            gather_wait(t)
            scatter_start(t)

    for m in range(NGB):
        scatter_wait(m)

    plsc.subcore_barrier()

    pltpu.sync_copy(accum.at[pl.ds(base, RA)],
                    out_hbm.at[c, pl.ds(base, RA)])

    @pl.when(s == 0)
    def _():
        pltpu.sync_copy(accum.at[pl.ds(NS * RA, TAIL)],
                        out_hbm.at[c, pl.ds(NS * RA, TAIL)])


def _out_body(a_ref, dinv_ref, w_ref, b_ref, o_ref):
    t = (a_ref[0] + a_ref[1]) * dinv_ref[...]
    acc = None
    for k in range(KW):
        f = jnp.dot(t, w_ref[k], preferred_element_type=jnp.float32)
        f = jnp.maximum(f + b_ref[k][None, :], 0.0)
        acc = f if acc is None else acc + f
    o_ref[...] = acc


_BN = 2000

_out = pl.pallas_call(
    _out_body,
    grid=(N // _BN,),
    in_specs=[
        pl.BlockSpec((NC, _BN, D), lambda i: (0, i, 0)),
        pl.BlockSpec((_BN, 1), lambda i: (i, 0)),
        pl.BlockSpec((KW, D, D), lambda i: (0, 0, 0)),
        pl.BlockSpec((KW, D), lambda i: (0, 0)),
    ],
    out_specs=pl.BlockSpec((_BN, D), lambda i: (i, 0)),
    out_shape=jax.ShapeDtypeStruct((N, D), jnp.float32),
)


@jax.jit
def kernel(x, edge_index, W, b):
    hists = _deg_kernel(edge_index).reshape(NW, N)
    y, dinv = _prep(hists, x)
    partials = _agg_kernel(y, edge_index)
    return _out(partials, dinv, W, b)
